# Initial kernel scaffold; baseline (speedup 1.0000x reference)
import jax
import jax.numpy as jnp
from jax.experimental import pallas as pl


def kernel(feature_vector, params):
    raise NotImplementedError("write your pallas kernel here")


# speedup vs baseline: 2.1086x; 2.1086x over previous
import jax
import jax.numpy as jnp
from jax.experimental import pallas as pl

_BATCH = 16384
_IN = 79
_EMB = 256
_NQ = 3
_K = 64
_CC = 0.25
_BB = 512


def _fused(fv_ref, encW_ref, encb_ref, g_ref, b_ref, decW_ref, decb_ref, cb_ref,
           enc_ref, q_ref, idx_ref, rec_ref, aq_ref, loss_ref):
    fv = fv_ref[...]
    h = jnp.dot(fv.astype(jnp.bfloat16), encW_ref[...].astype(jnp.bfloat16),
                preferred_element_type=jnp.float32) + encb_ref[...]
    mu = jnp.mean(h, axis=-1, keepdims=True)
    var = jnp.mean((h - mu) * (h - mu), axis=-1, keepdims=True)
    h = (h - mu) / jnp.sqrt(var + 1e-5) * g_ref[...] + b_ref[...]
    enc = 0.5 * h * (1.0 + jax.lax.erf(h * 0.7071067811865476))
    enc_ref[...] = enc

    residual = enc
    q_total = jnp.zeros_like(enc)
    loss_sum = jnp.float32(0.0)
    idx_levels = []
    for q in range(_NQ):
        cb = cb_ref[q]
        r2 = jnp.sum(residual * residual, axis=-1, keepdims=True)
        rc = jax.lax.dot_general(residual.astype(jnp.bfloat16), cb.astype(jnp.bfloat16),
                                 (((1,), (1,)), ((), ())),
                                 preferred_element_type=jnp.float32)
        c2 = jnp.sum(cb * cb, axis=-1)[None, :]
        d = r2 - 2.0 * rc + c2
        idx = jnp.argmin(d, axis=-1)
        onehot = (idx[:, None] == jax.lax.broadcasted_iota(jnp.int32, (1, _K), 1)
                  ).astype(jnp.float32)
        qv = jnp.dot(onehot, cb, preferred_element_type=jnp.float32, precision=jax.lax.Precision.HIGHEST)
        loss_sum = loss_sum + jnp.sum((residual - qv) * (residual - qv))
        aq_ref[:, q * _EMB:(q + 1) * _EMB] = qv
        q_total = q_total + qv
        residual = residual - qv
        idx_levels.append(idx)
    q_ref[...] = q_total
    rec_ref[...] = (jnp.dot(q_total.astype(jnp.bfloat16), decW_ref[...].astype(jnp.bfloat16),
                            preferred_element_type=jnp.float32)
                    + decb_ref[...])

    col = jax.lax.broadcasted_iota(jnp.int32, (_BB, _NQ), 1)
    i0 = idx_levels[0][:, None]
    i1 = idx_levels[1][:, None]
    i2 = idx_levels[2][:, None]
    idx_ref[...] = jnp.where(col == 0, i0, jnp.where(col == 1, i1, i2))

    @pl.when(pl.program_id(0) == 0)
    def _init():
        loss_ref[...] = jnp.zeros_like(loss_ref)

    loss_ref[...] = loss_ref[...] + loss_sum.reshape(1, 1)


def _run(fv, encW, encb, g, b, decW, decb, cb, interpret=False):
    nblk = _BATCH // _BB
    row = lambda i: (i, 0)
    fixed = lambda i: (0, 0)
    return pl.pallas_call(
        _fused,
        grid=(nblk,),
        in_specs=[
            pl.BlockSpec((_BB, _IN), row),
            pl.BlockSpec((_IN, _EMB), fixed),
            pl.BlockSpec((1, _EMB), fixed),
            pl.BlockSpec((1, _EMB), fixed),
            pl.BlockSpec((1, _EMB), fixed),
            pl.BlockSpec((_EMB, _IN), fixed),
            pl.BlockSpec((1, _IN), fixed),
            pl.BlockSpec((_NQ, _K, _EMB), lambda i: (0, 0, 0)),
        ],
        out_specs=[
            pl.BlockSpec((_BB, _EMB), row),
            pl.BlockSpec((_BB, _EMB), row),
            pl.BlockSpec((_BB, _NQ), row),
            pl.BlockSpec((_BB, _IN), row),
            pl.BlockSpec((_BB, _NQ * _EMB), row),
            pl.BlockSpec((1, 1), fixed),
        ],
        out_shape=[
            jax.ShapeDtypeStruct((_BATCH, _EMB), jnp.float32),
            jax.ShapeDtypeStruct((_BATCH, _EMB), jnp.float32),
            jax.ShapeDtypeStruct((_BATCH, _NQ), jnp.int32),
            jax.ShapeDtypeStruct((_BATCH, _IN), jnp.float32),
            jax.ShapeDtypeStruct((_BATCH, _NQ * _EMB), jnp.float32),
            jax.ShapeDtypeStruct((1, 1), jnp.float32),
        ],
        interpret=interpret,
    )(fv, encW, encb, g, b, decW, decb, cb)


def kernel(feature_vector, params):
    p = params
    enc, qt, idx, rec, aq2d, loss_sum = _run(
        feature_vector,
        p['enc_W'],
        p['enc_b'].reshape(1, _EMB),
        p['enc_ln_g'].reshape(1, _EMB),
        p['enc_ln_b'].reshape(1, _EMB),
        p['dec_W'],
        p['dec_b'].reshape(1, _IN),
        p['codebooks'],
    )
    loss = loss_sum[0, 0] * (_CC / (_BATCH * _EMB))
    all_q = aq2d.reshape(_BATCH, _NQ, _EMB)
    return (feature_vector, enc, qt, idx, rec, loss, all_q)


# speedup vs baseline: 2.2883x; 1.0852x over previous
import jax
import jax.numpy as jnp
from jax.experimental import pallas as pl

_BATCH = 16384
_IN = 79
_EMB = 256
_NQ = 3
_K = 64
_CC = 0.25
_BB = 1024


def _fused(fv_ref, encW_ref, encb_ref, g_ref, b_ref, decW_ref, decb_ref, cb_ref,
           enc_ref, q_ref, idx_ref, rec_ref, aq_ref, loss_ref):
    fv = fv_ref[...]
    h = jnp.dot(fv.astype(jnp.bfloat16), encW_ref[...].astype(jnp.bfloat16),
                preferred_element_type=jnp.float32) + encb_ref[...]
    mu = jnp.mean(h, axis=-1, keepdims=True)
    var = jnp.mean((h - mu) * (h - mu), axis=-1, keepdims=True)
    h = (h - mu) / jnp.sqrt(var + 1e-5) * g_ref[...] + b_ref[...]
    enc = 0.5 * h * (1.0 + jax.lax.erf(h * 0.7071067811865476))
    enc_ref[...] = enc

    residual = enc
    q_total = jnp.zeros_like(enc)
    loss_sum = jnp.float32(0.0)
    idx_levels = []
    for q in range(_NQ):
        cb = cb_ref[q]
        r2 = jnp.sum(residual * residual, axis=-1, keepdims=True)
        rc = jax.lax.dot_general(residual.astype(jnp.bfloat16), cb.astype(jnp.bfloat16),
                                 (((1,), (1,)), ((), ())),
                                 preferred_element_type=jnp.float32)
        c2 = jnp.sum(cb * cb, axis=-1)[None, :]
        d = r2 - 2.0 * rc + c2
        idx = jnp.argmin(d, axis=-1)
        onehot = (idx[:, None] == jax.lax.broadcasted_iota(jnp.int32, (1, _K), 1)
                  ).astype(jnp.float32)
        qv = jnp.dot(onehot, cb, preferred_element_type=jnp.float32, precision=jax.lax.Precision.HIGHEST)
        loss_sum = loss_sum + jnp.sum((residual - qv) * (residual - qv))
        aq_ref[:, q * _EMB:(q + 1) * _EMB] = qv
        q_total = q_total + qv
        residual = residual - qv
        idx_levels.append(idx)
    q_ref[...] = q_total
    rec_ref[...] = (jnp.dot(q_total.astype(jnp.bfloat16), decW_ref[...].astype(jnp.bfloat16),
                            preferred_element_type=jnp.float32)
                    + decb_ref[...])

    col = jax.lax.broadcasted_iota(jnp.int32, (_BB, _NQ), 1)
    i0 = idx_levels[0][:, None]
    i1 = idx_levels[1][:, None]
    i2 = idx_levels[2][:, None]
    idx_ref[...] = jnp.where(col == 0, i0, jnp.where(col == 1, i1, i2))

    @pl.when(pl.program_id(0) == 0)
    def _init():
        loss_ref[...] = jnp.zeros_like(loss_ref)

    loss_ref[...] = loss_ref[...] + loss_sum.reshape(1, 1)


def _run(fv, encW, encb, g, b, decW, decb, cb, interpret=False):
    nblk = _BATCH // _BB
    row = lambda i: (i, 0)
    fixed = lambda i: (0, 0)
    return pl.pallas_call(
        _fused,
        grid=(nblk,),
        in_specs=[
            pl.BlockSpec((_BB, _IN), row),
            pl.BlockSpec((_IN, _EMB), fixed),
            pl.BlockSpec((1, _EMB), fixed),
            pl.BlockSpec((1, _EMB), fixed),
            pl.BlockSpec((1, _EMB), fixed),
            pl.BlockSpec((_EMB, _IN), fixed),
            pl.BlockSpec((1, _IN), fixed),
            pl.BlockSpec((_NQ, _K, _EMB), lambda i: (0, 0, 0)),
        ],
        out_specs=[
            pl.BlockSpec((_BB, _EMB), row),
            pl.BlockSpec((_BB, _EMB), row),
            pl.BlockSpec((_BB, _NQ), row),
            pl.BlockSpec((_BB, _IN), row),
            pl.BlockSpec((_BB, _NQ * _EMB), row),
            pl.BlockSpec((1, 1), fixed),
        ],
        out_shape=[
            jax.ShapeDtypeStruct((_BATCH, _EMB), jnp.float32),
            jax.ShapeDtypeStruct((_BATCH, _EMB), jnp.float32),
            jax.ShapeDtypeStruct((_BATCH, _NQ), jnp.int32),
            jax.ShapeDtypeStruct((_BATCH, _IN), jnp.float32),
            jax.ShapeDtypeStruct((_BATCH, _NQ * _EMB), jnp.float32),
            jax.ShapeDtypeStruct((1, 1), jnp.float32),
        ],
        interpret=interpret,
    )(fv, encW, encb, g, b, decW, decb, cb)


def kernel(feature_vector, params):
    p = params
    enc, qt, idx, rec, aq2d, loss_sum = _run(
        feature_vector,
        p['enc_W'],
        p['enc_b'].reshape(1, _EMB),
        p['enc_ln_g'].reshape(1, _EMB),
        p['enc_ln_b'].reshape(1, _EMB),
        p['dec_W'],
        p['dec_b'].reshape(1, _IN),
        p['codebooks'],
    )
    loss = loss_sum[0, 0] * (_CC / (_BATCH * _EMB))
    all_q = aq2d.reshape(_BATCH, _NQ, _EMB)
    return (feature_vector, enc, qt, idx, rec, loss, all_q)


# speedup vs baseline: 2.3411x; 1.0231x over previous
import jax
import jax.numpy as jnp
from jax.experimental import pallas as pl

_BATCH = 16384
_IN = 79
_EMB = 256
_NQ = 3
_K = 64
_CC = 0.25
_BB = 2048


def _fused(fv_ref, encW_ref, encb_ref, g_ref, b_ref, decW_ref, decb_ref, cb_ref,
           enc_ref, q_ref, idx_ref, rec_ref, aq_ref, loss_ref):
    fv = fv_ref[...]
    h = jnp.dot(fv.astype(jnp.bfloat16), encW_ref[...].astype(jnp.bfloat16),
                preferred_element_type=jnp.float32) + encb_ref[...]
    mu = jnp.mean(h, axis=-1, keepdims=True)
    var = jnp.mean((h - mu) * (h - mu), axis=-1, keepdims=True)
    h = (h - mu) / jnp.sqrt(var + 1e-5) * g_ref[...] + b_ref[...]
    enc = 0.5 * h * (1.0 + jax.lax.erf(h * 0.7071067811865476))
    enc_ref[...] = enc

    residual = enc
    q_total = jnp.zeros_like(enc)
    loss_sum = jnp.float32(0.0)
    idx_levels = []
    for q in range(_NQ):
        cb = cb_ref[q]
        r2 = jnp.sum(residual * residual, axis=-1, keepdims=True)
        rc = jax.lax.dot_general(residual.astype(jnp.bfloat16), cb.astype(jnp.bfloat16),
                                 (((1,), (1,)), ((), ())),
                                 preferred_element_type=jnp.float32)
        c2 = jnp.sum(cb * cb, axis=-1)[None, :]
        d = r2 - 2.0 * rc + c2
        idx = jnp.argmin(d, axis=-1)
        onehot = (idx[:, None] == jax.lax.broadcasted_iota(jnp.int32, (1, _K), 1)
                  ).astype(jnp.float32)
        qv = jnp.dot(onehot, cb, preferred_element_type=jnp.float32, precision=jax.lax.Precision.HIGHEST)
        loss_sum = loss_sum + jnp.sum((residual - qv) * (residual - qv))
        aq_ref[:, q * _EMB:(q + 1) * _EMB] = qv
        q_total = q_total + qv
        residual = residual - qv
        idx_levels.append(idx)
    q_ref[...] = q_total
    rec_ref[...] = (jnp.dot(q_total.astype(jnp.bfloat16), decW_ref[...].astype(jnp.bfloat16),
                            preferred_element_type=jnp.float32)
                    + decb_ref[...])

    col = jax.lax.broadcasted_iota(jnp.int32, (_BB, _NQ), 1)
    i0 = idx_levels[0][:, None]
    i1 = idx_levels[1][:, None]
    i2 = idx_levels[2][:, None]
    idx_ref[...] = jnp.where(col == 0, i0, jnp.where(col == 1, i1, i2))

    @pl.when(pl.program_id(0) == 0)
    def _init():
        loss_ref[...] = jnp.zeros_like(loss_ref)

    loss_ref[...] = loss_ref[...] + loss_sum.reshape(1, 1)


def _run(fv, encW, encb, g, b, decW, decb, cb, interpret=False):
    nblk = _BATCH // _BB
    row = lambda i: (i, 0)
    fixed = lambda i: (0, 0)
    return pl.pallas_call(
        _fused,
        grid=(nblk,),
        in_specs=[
            pl.BlockSpec((_BB, _IN), row),
            pl.BlockSpec((_IN, _EMB), fixed),
            pl.BlockSpec((1, _EMB), fixed),
            pl.BlockSpec((1, _EMB), fixed),
            pl.BlockSpec((1, _EMB), fixed),
            pl.BlockSpec((_EMB, _IN), fixed),
            pl.BlockSpec((1, _IN), fixed),
            pl.BlockSpec((_NQ, _K, _EMB), lambda i: (0, 0, 0)),
        ],
        out_specs=[
            pl.BlockSpec((_BB, _EMB), row),
            pl.BlockSpec((_BB, _EMB), row),
            pl.BlockSpec((_BB, _NQ), row),
            pl.BlockSpec((_BB, _IN), row),
            pl.BlockSpec((_BB, _NQ * _EMB), row),
            pl.BlockSpec((1, 1), fixed),
        ],
        out_shape=[
            jax.ShapeDtypeStruct((_BATCH, _EMB), jnp.float32),
            jax.ShapeDtypeStruct((_BATCH, _EMB), jnp.float32),
            jax.ShapeDtypeStruct((_BATCH, _NQ), jnp.int32),
            jax.ShapeDtypeStruct((_BATCH, _IN), jnp.float32),
            jax.ShapeDtypeStruct((_BATCH, _NQ * _EMB), jnp.float32),
            jax.ShapeDtypeStruct((1, 1), jnp.float32),
        ],
        interpret=interpret,
    )(fv, encW, encb, g, b, decW, decb, cb)


def kernel(feature_vector, params):
    p = params
    enc, qt, idx, rec, aq2d, loss_sum = _run(
        feature_vector,
        p['enc_W'],
        p['enc_b'].reshape(1, _EMB),
        p['enc_ln_g'].reshape(1, _EMB),
        p['enc_ln_b'].reshape(1, _EMB),
        p['dec_W'],
        p['dec_b'].reshape(1, _IN),
        p['codebooks'],
    )
    loss = loss_sum[0, 0] * (_CC / (_BATCH * _EMB))
    all_q = aq2d.reshape(_BATCH, _NQ, _EMB)
    return (feature_vector, enc, qt, idx, rec, loss, all_q)


# speedup vs baseline: 2.7274x; 1.1650x over previous
import jax
import jax.numpy as jnp
from jax.experimental import pallas as pl

_BATCH = 16384
_IN = 79
_EMB = 256
_NQ = 3
_K = 64
_CC = 0.25
_BB = 2048


def _fused(fv_ref, encW_ref, encb_ref, g_ref, b_ref, decW_ref, decb_ref, cb_ref,
           enc_ref, q_ref, idx_ref, rec_ref, aq_ref, loss_ref):
    fv = fv_ref[...]
    h = jnp.dot(fv.astype(jnp.bfloat16), encW_ref[...].astype(jnp.bfloat16),
                preferred_element_type=jnp.float32) + encb_ref[...]
    mu = jnp.mean(h, axis=-1, keepdims=True)
    var = jnp.mean((h - mu) * (h - mu), axis=-1, keepdims=True)
    h = (h - mu) / jnp.sqrt(var + 1e-5) * g_ref[...] + b_ref[...]
    enc = 0.5 * h * (1.0 + jax.lax.erf(h * 0.7071067811865476))
    enc_ref[...] = enc

    residual = enc
    q_total = jnp.zeros_like(enc)
    loss_sum = jnp.float32(0.0)
    idx_levels = []
    r2 = jnp.sum(residual * residual, axis=-1, keepdims=True)
    for q in range(_NQ):
        cb = cb_ref[q]
        cb_hi = cb.astype(jnp.bfloat16)
        res1 = cb - cb_hi.astype(jnp.float32)
        cb_mid = res1.astype(jnp.bfloat16)
        cb_lo = (res1 - cb_mid.astype(jnp.float32)).astype(jnp.bfloat16)
        rc = jax.lax.dot_general(residual.astype(jnp.bfloat16), cb_hi,
                                 (((1,), (1,)), ((), ())),
                                 preferred_element_type=jnp.float32)
        c2 = jnp.sum(cb * cb, axis=-1)[None, :]
        d = r2 - 2.0 * rc + c2
        idx = jnp.argmin(d, axis=-1)
        onehot = (idx[:, None] == jax.lax.broadcasted_iota(jnp.int32, (1, _K), 1)
                  ).astype(jnp.bfloat16)
        qv = ((jnp.dot(onehot, cb_hi, preferred_element_type=jnp.float32)
               + jnp.dot(onehot, cb_mid, preferred_element_type=jnp.float32))
              + jnp.dot(onehot, cb_lo, preferred_element_type=jnp.float32))
        aq_ref[:, q * _EMB:(q + 1) * _EMB] = qv
        q_total = q_total + qv
        residual = residual - qv
        r2 = jnp.sum(residual * residual, axis=-1, keepdims=True)
        loss_sum = loss_sum + jnp.sum(r2)
        idx_levels.append(idx)
    q_ref[...] = q_total
    rec_ref[...] = (jnp.dot(q_total.astype(jnp.bfloat16), decW_ref[...].astype(jnp.bfloat16),
                            preferred_element_type=jnp.float32)
                    + decb_ref[...])

    col = jax.lax.broadcasted_iota(jnp.int32, (_BB, _NQ), 1)
    i0 = idx_levels[0][:, None]
    i1 = idx_levels[1][:, None]
    i2 = idx_levels[2][:, None]
    idx_ref[...] = jnp.where(col == 0, i0, jnp.where(col == 1, i1, i2))

    @pl.when(pl.program_id(0) == 0)
    def _init():
        loss_ref[...] = jnp.zeros_like(loss_ref)

    loss_ref[...] = loss_ref[...] + loss_sum.reshape(1, 1)


def _run(fv, encW, encb, g, b, decW, decb, cb, interpret=False):
    nblk = _BATCH // _BB
    row = lambda i: (i, 0)
    fixed = lambda i: (0, 0)
    return pl.pallas_call(
        _fused,
        grid=(nblk,),
        in_specs=[
            pl.BlockSpec((_BB, _IN), row),
            pl.BlockSpec((_IN, _EMB), fixed),
            pl.BlockSpec((1, _EMB), fixed),
            pl.BlockSpec((1, _EMB), fixed),
            pl.BlockSpec((1, _EMB), fixed),
            pl.BlockSpec((_EMB, _IN), fixed),
            pl.BlockSpec((1, _IN), fixed),
            pl.BlockSpec((_NQ, _K, _EMB), lambda i: (0, 0, 0)),
        ],
        out_specs=[
            pl.BlockSpec((_BB, _EMB), row),
            pl.BlockSpec((_BB, _EMB), row),
            pl.BlockSpec((_BB, _NQ), row),
            pl.BlockSpec((_BB, _IN), row),
            pl.BlockSpec((_BB, _NQ * _EMB), row),
            pl.BlockSpec((1, 1), fixed),
        ],
        out_shape=[
            jax.ShapeDtypeStruct((_BATCH, _EMB), jnp.float32),
            jax.ShapeDtypeStruct((_BATCH, _EMB), jnp.float32),
            jax.ShapeDtypeStruct((_BATCH, _NQ), jnp.int32),
            jax.ShapeDtypeStruct((_BATCH, _IN), jnp.float32),
            jax.ShapeDtypeStruct((_BATCH, _NQ * _EMB), jnp.float32),
            jax.ShapeDtypeStruct((1, 1), jnp.float32),
        ],
        interpret=interpret,
    )(fv, encW, encb, g, b, decW, decb, cb)


def kernel(feature_vector, params):
    p = params
    enc, qt, idx, rec, aq2d, loss_sum = _run(
        feature_vector,
        p['enc_W'],
        p['enc_b'].reshape(1, _EMB),
        p['enc_ln_g'].reshape(1, _EMB),
        p['enc_ln_b'].reshape(1, _EMB),
        p['dec_W'],
        p['dec_b'].reshape(1, _IN),
        p['codebooks'],
    )
    loss = loss_sum[0, 0] * (_CC / (_BATCH * _EMB))
    all_q = aq2d.reshape(_BATCH, _NQ, _EMB)
    return (feature_vector, enc, qt, idx, rec, loss, all_q)
